# baseline (speedup 1.0000x reference)
import functools

import jax
import jax.numpy as jnp
from jax import lax
from jax.experimental import pallas as pl
from jax.experimental.pallas import tpu as pltpu
from jax.experimental.pallas import tpu_sc as plsc

NC = 2
NS = 16
LANES = 16
CHUNK = 64


def _sc_spmm(inp, src2d, dst2d, vals, *, n, d, e_per_tile):
    n_stages = 4
    stage = e_per_tile // n_stages
    stage_rows = stage // CHUNK
    n_quads = stage // (4 * CHUNK)
    tile_rows = e_per_tile // CHUNK
    rows_per = (n // NS) // 8 * 8
    row_rem = n - rows_per * NS

    mesh = plsc.VectorSubcoreMesh(
        core_axis_name="c", subcore_axis_name="s",
        num_cores=NC, num_subcores=NS)

    @functools.partial(
        pl.kernel,
        out_type=jax.ShapeDtypeStruct((NC, n, d), jnp.float32),
        mesh=mesh,
        scratch_types=[
            pltpu.VMEM((stage_rows, CHUNK), jnp.int32),
            pltpu.VMEM((stage_rows, CHUNK), jnp.int32),
            pltpu.VMEM((stage,), jnp.float32),
            pltpu.VMEM((CHUNK, d), jnp.float32),
            pltpu.VMEM((CHUNK, d), jnp.float32),
            pltpu.VMEM((CHUNK, d), jnp.float32),
            pltpu.VMEM((CHUNK, d), jnp.float32),
            pltpu.VMEM_SHARED((n, d), jnp.float32),
            pltpu.SemaphoreType.DMA,
            pltpu.SemaphoreType.DMA,
            pltpu.SemaphoreType.DMA,
            pltpu.SemaphoreType.DMA,
            pltpu.SemaphoreType.DMA,
            pltpu.SemaphoreType.DMA,
        ],
    )
    def spmm(inp_hbm, src_hbm, dst_hbm, val_hbm, out_hbm,
             srcv, dstv, valv, buf_a, buf_b, buf_c, buf_d, acc,
             gsem_a, gsem_b, gsem_c, gsem_d, ssem, ssem_d):
        cid = lax.axis_index("c")
        sid = lax.axis_index("s")
        tile = cid * NS + sid
        bufs = (buf_a, buf_b, buf_c, buf_d)
        gsems = (gsem_a, gsem_b, gsem_c, gsem_d)

        zbase = sid * rows_per
        zero16 = jnp.zeros((LANES,), jnp.float32)

        def fill_buf(r, c2):
            for k in range(d // LANES):
                buf_a[r, pl.ds(k * LANES, LANES)] = zero16
            return c2
        lax.fori_loop(0, CHUNK, fill_buf, 0)

        n_zcopies = rows_per // CHUNK
        zrem = rows_per - n_zcopies * CHUNK

        def zcopy(i, c2):
            pltpu.sync_copy(buf_a.at[pl.ds(0, CHUNK)],
                            acc.at[pl.ds(zbase + i * CHUNK, CHUNK)])
            return c2
        lax.fori_loop(0, n_zcopies, zcopy, 0)
        if zrem:
            pltpu.sync_copy(buf_a.at[pl.ds(0, zrem)],
                            acc.at[pl.ds(zbase + n_zcopies * CHUNK, zrem)])
        if row_rem:
            @pl.when(sid == NS - 1)
            def _():
                pltpu.sync_copy(buf_a.at[pl.ds(0, row_rem)],
                                acc.at[pl.ds(NS * rows_per, row_rem)])
        plsc.subcore_barrier()

        def gather(r, t):
            pltpu.async_copy(inp_hbm.at[srcv.at[r]], bufs[t], gsems[t])

        def wait_gather(t):
            pltpu.make_async_copy(inp_hbm.at[srcv.at[0]], bufs[t],
                                  gsems[t]).wait()

        def scatter(r, t, sem):
            return pltpu.async_copy(bufs[t], acc.at[dstv.at[r]], sem,
                                    add=True)

        def wait_scatter_d():
            pltpu.make_async_copy(buf_d, acc.at[dstv.at[0]], ssem_d).wait()

        def scale(t, coff):
            buf = bufs[t]

            def q_body(q, c2):
                vvec = valv[pl.ds(coff * CHUNK + q * LANES, LANES)]
                for j in range(LANES):
                    v = vvec[j]
                    e_idx = q * LANES + j
                    for k in range(d // LANES):
                        sl = pl.ds(k * LANES, LANES)
                        buf[e_idx, sl] = buf[e_idx, sl] * v
                return c2
            lax.fori_loop(0, CHUNK // LANES, q_body, 0)

        def quad(base, *, first, last):
            if not first:
                wait_scatter_d()
                gather(base + 3, 3)
            wait_gather(0)
            scale(0, base + 0)
            s_a = scatter(base + 0, 0, ssem)
            wait_gather(1)
            scale(1, base + 1)
            s_b = scatter(base + 1, 1, ssem)
            s_a.wait()
            if not last:
                gather(base + 4, 0)
            wait_gather(2)
            scale(2, base + 2)
            s_c = scatter(base + 2, 2, ssem)
            s_b.wait()
            if not last:
                gather(base + 5, 1)
            wait_gather(3)
            scale(3, base + 3)
            s_d = scatter(base + 3, 3, ssem_d)
            s_c.wait()
            if not last:
                gather(base + 6, 2)
            if last:
                s_d.wait()
            return s_d

        def stage_body(s, carry):
            r0 = tile * tile_rows + s * stage_rows
            e0 = (tile * n_stages + s) * stage
            pltpu.sync_copy(src_hbm.at[pl.ds(r0, stage_rows)], srcv)
            pltpu.sync_copy(dst_hbm.at[pl.ds(r0, stage_rows)], dstv)
            pltpu.sync_copy(val_hbm.at[pl.ds(e0, stage)], valv)
            gather(0, 0)
            gather(1, 1)
            gather(2, 2)
            gather(3, 3)
            quad(0, first=True, last=False)

            def steady(i, c2):
                quad(4 * i, first=False, last=False)
                return c2
            lax.fori_loop(1, n_quads - 1, steady, 0)
            quad(4 * (n_quads - 1), first=False, last=True)
            return carry

        lax.fori_loop(0, n_stages, stage_body, 0)

        plsc.subcore_barrier()
        pltpu.sync_copy(acc.at[pl.ds(zbase, rows_per)],
                        out_hbm.at[cid, pl.ds(zbase, rows_per)])
        if row_rem:
            @pl.when(sid == NS - 1)
            def _():
                pltpu.sync_copy(acc.at[pl.ds(NS * rows_per, row_rem)],
                                out_hbm.at[cid, pl.ds(NS * rows_per, row_rem)])

    return spmm(inp, src2d, dst2d, vals)


def _combine_body(a_ref, p_ref, x_ref, o_ref):
    a1 = a_ref[0, 0]
    a2 = a_ref[0, 1]
    o_ref[...] = a1 * (p_ref[0] + p_ref[1]) + a2 * x_ref[...]


def _tc_combine(partials, x, alphas, *, n, d, block_rows):
    grid = n // block_rows
    return pl.pallas_call(
        _combine_body,
        grid=(grid,),
        in_specs=[
            pl.BlockSpec(memory_space=pltpu.SMEM),
            pl.BlockSpec((NC, block_rows, d), lambda i: (0, i, 0)),
            pl.BlockSpec((block_rows, d), lambda i: (i, 0)),
        ],
        out_specs=pl.BlockSpec((block_rows, d), lambda i: (i, 0)),
        out_shape=jax.ShapeDtypeStruct((n, d), jnp.float32),
    )(alphas, partials, x)


def kernel(inp, adj_indices, adj_values, x, alpha1, alpha2):
    n, d = inp.shape
    e = adj_values.shape[0]

    grain = NC * NS * 2048
    e_pad = ((e + grain - 1) // grain) * grain
    pad = e_pad - e

    if pad:
        pad_idx = (jnp.arange(pad, dtype=jnp.int32) % n).astype(jnp.int32)
        idx = jnp.concatenate(
            [adj_indices, jnp.broadcast_to(pad_idx, (2, pad))], axis=1)
        vals = jnp.concatenate(
            [adj_values, jnp.zeros((pad,), dtype=jnp.float32)])
    else:
        idx = adj_indices
        vals = adj_values
    idx3d = idx.reshape(2, e_pad // CHUNK, CHUNK)
    src2d = idx3d[1]
    dst2d = idx3d[0]

    partials = _sc_spmm(inp, src2d, dst2d, vals,
                        n=n, d=d, e_per_tile=e_pad // (NC * NS))

    alphas = jnp.concatenate([alpha1, alpha2]).reshape(1, 2)
    block_rows = 2000 if n % 2000 == 0 else n
    return _tc_combine(partials, x, alphas, n=n, d=d, block_rows=block_rows)


# speedup vs baseline: 1.2055x; 1.2055x over previous
import functools

import jax
import jax.numpy as jnp
from jax import lax
from jax.experimental import pallas as pl
from jax.experimental.pallas import tpu as pltpu
from jax.experimental.pallas import tpu_sc as plsc

NC = 2
NS = 16
LANES = 16
CHUNK = 64


def _sc_spmm(inp, src2d, dst2d, vals, *, n, d, e_per_tile):
    n_stages = 4
    stage = e_per_tile // n_stages
    stage_rows = stage // CHUNK
    n_quads = stage // (4 * CHUNK)
    tile_rows = e_per_tile // CHUNK
    rows_per = (n // NS) // 8 * 8
    row_rem = n - rows_per * NS

    mesh = plsc.VectorSubcoreMesh(
        core_axis_name="c", subcore_axis_name="s",
        num_cores=NC, num_subcores=NS)

    @functools.partial(
        pl.kernel,
        out_type=jax.ShapeDtypeStruct((NC, n, d), jnp.float32),
        mesh=mesh,
        scratch_types=[
            pltpu.VMEM((stage_rows, CHUNK), jnp.int32),
            pltpu.VMEM((stage_rows, CHUNK), jnp.int32),
            pltpu.VMEM((stage,), jnp.float32),
            pltpu.VMEM((CHUNK, d), jnp.float32),
            pltpu.VMEM((CHUNK, d), jnp.float32),
            pltpu.VMEM((CHUNK, d), jnp.float32),
            pltpu.VMEM((CHUNK, d), jnp.float32),
            pltpu.VMEM_SHARED((n, d), jnp.float32),
            pltpu.SemaphoreType.DMA,
            pltpu.SemaphoreType.DMA,
            pltpu.SemaphoreType.DMA,
            pltpu.SemaphoreType.DMA,
            pltpu.SemaphoreType.DMA,
            pltpu.SemaphoreType.DMA,
        ],
    )
    def spmm(inp_hbm, src_hbm, dst_hbm, val_hbm, out_hbm,
             srcv, dstv, valv, buf_a, buf_b, buf_c, buf_d, acc,
             gsem_a, gsem_b, gsem_c, gsem_d, ssem, ssem_d):
        cid = lax.axis_index("c")
        sid = lax.axis_index("s")
        tile = cid * NS + sid
        bufs = (buf_a, buf_b, buf_c, buf_d)
        gsems = (gsem_a, gsem_b, gsem_c, gsem_d)

        zbase = sid * rows_per
        zero16 = jnp.zeros((LANES,), jnp.float32)

        def fill_buf(r, c2):
            for k in range(d // LANES):
                buf_a[r, pl.ds(k * LANES, LANES)] = zero16
            return c2
        lax.fori_loop(0, CHUNK, fill_buf, 0)

        n_zcopies = rows_per // CHUNK
        zrem = rows_per - n_zcopies * CHUNK

        def zcopy(i, c2):
            pltpu.sync_copy(buf_a.at[pl.ds(0, CHUNK)],
                            acc.at[pl.ds(zbase + i * CHUNK, CHUNK)])
            return c2
        lax.fori_loop(0, n_zcopies, zcopy, 0)
        if zrem:
            pltpu.sync_copy(buf_a.at[pl.ds(0, zrem)],
                            acc.at[pl.ds(zbase + n_zcopies * CHUNK, zrem)])
        if row_rem:
            @pl.when(sid == NS - 1)
            def _():
                pltpu.sync_copy(buf_a.at[pl.ds(0, row_rem)],
                                acc.at[pl.ds(NS * rows_per, row_rem)])
        plsc.subcore_barrier()

        def gather(r, t):
            pltpu.async_copy(inp_hbm.at[srcv.at[r]], bufs[t], gsems[t])

        def wait_gather(t):
            pltpu.make_async_copy(inp_hbm.at[srcv.at[0]], bufs[t],
                                  gsems[t]).wait()

        def scatter(r, t, sem):
            return pltpu.async_copy(bufs[t], acc.at[dstv.at[r]], sem,
                                    add=True)

        def wait_scatter_d():
            pltpu.make_async_copy(buf_d, acc.at[dstv.at[0]], ssem_d).wait()

        def scale(t, coff):
            buf = bufs[t]

            @functools.partial(plsc.parallel_loop, 0, CHUNK // LANES)
            def q_body(q):
                vvec = valv[pl.ds(coff * CHUNK + q * LANES, LANES)]
                for j in range(LANES):
                    v = vvec[j]
                    e_idx = q * LANES + j
                    for k in range(d // LANES):
                        sl = pl.ds(k * LANES, LANES)
                        buf[e_idx, sl] = buf[e_idx, sl] * v

        def quad(base, *, first, last):
            if not first:
                wait_scatter_d()
                gather(base + 3, 3)
            wait_gather(0)
            scale(0, base + 0)
            s_a = scatter(base + 0, 0, ssem)
            wait_gather(1)
            scale(1, base + 1)
            s_b = scatter(base + 1, 1, ssem)
            s_a.wait()
            if not last:
                gather(base + 4, 0)
            wait_gather(2)
            scale(2, base + 2)
            s_c = scatter(base + 2, 2, ssem)
            s_b.wait()
            if not last:
                gather(base + 5, 1)
            wait_gather(3)
            scale(3, base + 3)
            s_d = scatter(base + 3, 3, ssem_d)
            s_c.wait()
            if not last:
                gather(base + 6, 2)
            if last:
                s_d.wait()
            return s_d

        def stage_body(s, carry):
            r0 = tile * tile_rows + s * stage_rows
            e0 = (tile * n_stages + s) * stage
            pltpu.sync_copy(src_hbm.at[pl.ds(r0, stage_rows)], srcv)
            pltpu.sync_copy(dst_hbm.at[pl.ds(r0, stage_rows)], dstv)
            pltpu.sync_copy(val_hbm.at[pl.ds(e0, stage)], valv)
            gather(0, 0)
            gather(1, 1)
            gather(2, 2)
            gather(3, 3)
            quad(0, first=True, last=False)

            def steady(i, c2):
                quad(4 * i, first=False, last=False)
                return c2
            lax.fori_loop(1, n_quads - 1, steady, 0)
            quad(4 * (n_quads - 1), first=False, last=True)
            return carry

        lax.fori_loop(0, n_stages, stage_body, 0)

        plsc.subcore_barrier()
        pltpu.sync_copy(acc.at[pl.ds(zbase, rows_per)],
                        out_hbm.at[cid, pl.ds(zbase, rows_per)])
        if row_rem:
            @pl.when(sid == NS - 1)
            def _():
                pltpu.sync_copy(acc.at[pl.ds(NS * rows_per, row_rem)],
                                out_hbm.at[cid, pl.ds(NS * rows_per, row_rem)])

    return spmm(inp, src2d, dst2d, vals)


def _combine_body(a_ref, p_ref, x_ref, o_ref):
    a1 = a_ref[0, 0]
    a2 = a_ref[0, 1]
    o_ref[...] = a1 * (p_ref[0] + p_ref[1]) + a2 * x_ref[...]


def _tc_combine(partials, x, alphas, *, n, d, block_rows):
    grid = n // block_rows
    return pl.pallas_call(
        _combine_body,
        grid=(grid,),
        in_specs=[
            pl.BlockSpec(memory_space=pltpu.SMEM),
            pl.BlockSpec((NC, block_rows, d), lambda i: (0, i, 0)),
            pl.BlockSpec((block_rows, d), lambda i: (i, 0)),
        ],
        out_specs=pl.BlockSpec((block_rows, d), lambda i: (i, 0)),
        out_shape=jax.ShapeDtypeStruct((n, d), jnp.float32),
    )(alphas, partials, x)


def kernel(inp, adj_indices, adj_values, x, alpha1, alpha2):
    n, d = inp.shape
    e = adj_values.shape[0]

    grain = NC * NS * 2048
    e_pad = ((e + grain - 1) // grain) * grain
    pad = e_pad - e

    dst = adj_indices[0]
    src = adj_indices[1]
    if pad:
        pad_idx = (jnp.arange(pad, dtype=jnp.int32) % n).astype(jnp.int32)
        src = jnp.concatenate([src, pad_idx])
        dst = jnp.concatenate([dst, pad_idx])
        vals = jnp.concatenate(
            [adj_values, jnp.zeros((pad,), dtype=jnp.float32)])
    else:
        vals = adj_values
    src2d = src.reshape(e_pad // CHUNK, CHUNK)
    dst2d = dst.reshape(e_pad // CHUNK, CHUNK)

    partials = _sc_spmm(inp, src2d, dst2d, vals,
                        n=n, d=d, e_per_tile=e_pad // (NC * NS))

    alphas = jnp.concatenate([alpha1, alpha2]).reshape(1, 2)
    block_rows = 2000 if n % 2000 == 0 else n
    return _tc_combine(partials, x, alphas, n=n, d=d, block_rows=block_rows)
